# baseline (speedup 1.0000x reference)
import functools

import jax
import jax.numpy as jnp
from jax import lax
from jax.experimental import pallas as pl
from jax.experimental.pallas import tpu as pltpu
from jax.experimental.pallas import tpu_sc as plsc

_VOCAB = 1000000
_EMSIZE = 64
_PADE = 128
_B = 4096
_L = 50

_NC = 2
_NS = 16
_NW = _NC * _NS
_RPW = _B // _NW
_NBUF = 5
_NROUND = _L // _NBUF

_mesh = plsc.VectorSubcoreMesh(core_axis_name="c", subcore_axis_name="s")

_TBS = 4096
_TGRID = -(-_VOCAB // _TBS)


def _tpose_body(t_ref, o_ref):
    eye = jnp.eye(_EMSIZE, dtype=jnp.float32)
    t = jax.lax.dot_general(
        t_ref[...], eye, (((0,), (0,)), ((), ())),
        preferred_element_type=jnp.float32)
    o_ref[...] = jnp.concatenate(
        [t, jnp.zeros((_TBS, _PADE - _EMSIZE), jnp.float32)], axis=1)


_transpose_pad = pl.pallas_call(
    _tpose_body,
    grid=(_TGRID,),
    in_specs=[pl.BlockSpec((_EMSIZE, _TBS), lambda k: (0, k))],
    out_specs=pl.BlockSpec((_TBS, _PADE), lambda k: (k, 0)),
    out_shape=jax.ShapeDtypeStruct((_VOCAB, _PADE), jnp.float32),
)


@functools.partial(
    pl.kernel,
    mesh=_mesh,
    compiler_params=pltpu.CompilerParams(
        use_tc_tiling_on_sc=False, needs_layout_passes=False),
    out_type=jax.ShapeDtypeStruct((_L, _EMSIZE, _B), jnp.float32),
    scratch_types=(
        [pltpu.VMEM((_L, _RPW), jnp.int32)]
        + [pltpu.VMEM((_RPW, _PADE), jnp.float32)] * _NBUF
        + [pltpu.VMEM((_EMSIZE, _RPW), jnp.float32)] * _NBUF
        + [pltpu.SemaphoreType.DMA] * (2 * _NBUF)
    ),
)
def _embed_sc(idx_hbm, table_hbm, out_hbm, idx_v, *rest):
    bufs = rest[:_NBUF]
    bufTs = rest[_NBUF:2 * _NBUF]
    gs = rest[2 * _NBUF:3 * _NBUF]
    ws = rest[3 * _NBUF:]
    wid = lax.axis_index("s") * _NC + lax.axis_index("c")
    base = wid * _RPW
    pltpu.sync_copy(idx_hbm.at[:, pl.ds(base, _RPW)], idx_v)

    def start_gather(b, j):
        pltpu.async_copy(table_hbm.at[idx_v.at[j]], bufs[b], gs[b])

    def wait_gather(b, j):
        pltpu.make_async_copy(
            table_hbm.at[idx_v.at[j]], bufs[b], gs[b]).wait()

    def start_wb(b, j):
        pltpu.async_copy(
            bufTs[b], out_hbm.at[j, :, pl.ds(base, _RPW)], ws[b])

    def wait_wb(b, j):
        pltpu.make_async_copy(
            bufTs[b], out_hbm.at[j, :, pl.ds(base, _RPW)], ws[b]).wait()

    riotas = [lax.iota(jnp.int32, 16) + 16 * k for k in range(_RPW // 16)]

    def transpose(b):
        def cbody(c, carry):
            col = jnp.full((16,), c, jnp.int32)
            for k in range(_RPW // 16):
                v = plsc.load_gather(bufs[b], [riotas[k], col])
                bufTs[b][c, pl.ds(16 * k, 16)] = v
            return carry
        lax.fori_loop(0, _EMSIZE, cbody, 0)

    for b in range(_NBUF):
        start_gather(b, b)
    for b in range(_NBUF):
        wait_gather(b, b)
        transpose(b)
        start_wb(b, b)
        start_gather(b, b + _NBUF)

    def round_body(g, carry):
        for b in range(_NBUF):
            j = g * _NBUF + b
            wait_gather(b, j)
            wait_wb(b, j - _NBUF)
            transpose(b)
            start_wb(b, j)
            start_gather(b, j + _NBUF)
        return carry

    lax.fori_loop(1, _NROUND - 1, round_body, 0)

    gl = _NROUND - 1
    for b in range(_NBUF):
        j = gl * _NBUF + b
        wait_gather(b, j)
        wait_wb(b, j - _NBUF)
        transpose(b)
        start_wb(b, j)
    for b in range(_NBUF):
        wait_wb(b, gl * _NBUF + b)


def kernel(input_variable, embedding_weight):
    idx = input_variable
    if idx.dtype != jnp.int32:
        idx = idx.astype(jnp.int32)
    table128 = _transpose_pad(embedding_weight.T)
    out3d = _embed_sc(idx.T, table128)
    return jnp.transpose(out3d, (2, 0, 1))


# speedup vs baseline: 1.0813x; 1.0813x over previous
import functools

import jax
import jax.numpy as jnp
from jax import lax
from jax.experimental import pallas as pl
from jax.experimental.pallas import tpu as pltpu
from jax.experimental.pallas import tpu_sc as plsc

_VOCAB = 1000000
_EMSIZE = 64
_PADE = 128
_B = 4096
_L = 50

_NC = 2
_NS = 16
_NW = _NC * _NS
_RPW = _B // _NW
_NBUF = 5
_NROUND = _L // _NBUF

_mesh = plsc.VectorSubcoreMesh(core_axis_name="c", subcore_axis_name="s")

_TBS = 4096
_TGRID = -(-_VOCAB // _TBS)


def _tpose_body(t_ref, o_ref):
    eye = jnp.eye(_EMSIZE, dtype=jnp.float32)
    t = jax.lax.dot_general(
        t_ref[...], eye, (((0,), (0,)), ((), ())),
        preferred_element_type=jnp.float32)
    o_ref[...] = jnp.concatenate(
        [t, jnp.zeros((_TBS, _PADE - _EMSIZE), jnp.float32)], axis=1)


_transpose_pad = pl.pallas_call(
    _tpose_body,
    grid=(_TGRID,),
    in_specs=[pl.BlockSpec((_EMSIZE, _TBS), lambda k: (0, k))],
    out_specs=pl.BlockSpec((_TBS, _PADE), lambda k: (k, 0)),
    out_shape=jax.ShapeDtypeStruct((_VOCAB, _PADE), jnp.float32),
)


@functools.partial(
    pl.kernel,
    mesh=_mesh,
    compiler_params=pltpu.CompilerParams(
        use_tc_tiling_on_sc=False, needs_layout_passes=False),
    out_type=jax.ShapeDtypeStruct((_L, _EMSIZE, _B), jnp.float32),
    scratch_types=(
        [pltpu.VMEM((_L, _RPW), jnp.int32)]
        + [pltpu.VMEM((_RPW, _PADE), jnp.float32)] * _NBUF
        + [pltpu.VMEM((_EMSIZE, _RPW), jnp.float32)] * _NBUF
        + [pltpu.SemaphoreType.DMA] * (2 * _NBUF)
    ),
)
def _embed_sc(idx_hbm, table_hbm, out_hbm, idx_v, *rest):
    bufs = rest[:_NBUF]
    bufTs = rest[_NBUF:2 * _NBUF]
    gs = rest[2 * _NBUF:3 * _NBUF]
    ws = rest[3 * _NBUF:]
    wid = lax.axis_index("s") * _NC + lax.axis_index("c")
    base = wid * _RPW
    pltpu.sync_copy(idx_hbm.at[:, pl.ds(base, _RPW)], idx_v)

    def start_gather(b, j):
        pltpu.async_copy(table_hbm.at[idx_v.at[j]], bufs[b], gs[b])

    def wait_gather(b, j):
        pltpu.make_async_copy(
            table_hbm.at[idx_v.at[j]], bufs[b], gs[b]).wait()

    def start_wb(b, j):
        pltpu.async_copy(
            bufTs[b], out_hbm.at[j, :, pl.ds(base, _RPW)], ws[b])

    def wait_wb(b, j):
        pltpu.make_async_copy(
            bufTs[b], out_hbm.at[j, :, pl.ds(base, _RPW)], ws[b]).wait()

    riotas = [lax.iota(jnp.int32, 16) + 16 * k for k in range(_RPW // 16)]

    def transpose(b):
        def cbody(ci, carry):
            c0 = ci * 4
            for cc in range(4):
                col = jnp.full((16,), c0 + cc, jnp.int32)
                vs = [plsc.load_gather(bufs[b], [riotas[k], col])
                      for k in range(_RPW // 16)]
                for k in range(_RPW // 16):
                    bufTs[b][c0 + cc, pl.ds(16 * k, 16)] = vs[k]
            return carry
        lax.fori_loop(0, _EMSIZE // 4, cbody, 0)

    for b in range(_NBUF):
        start_gather(b, b)
    for b in range(_NBUF):
        wait_gather(b, b)
        transpose(b)
        start_wb(b, b)
        start_gather(b, b + _NBUF)

    def round_body(g, carry):
        for b in range(_NBUF):
            j = g * _NBUF + b
            wait_gather(b, j)
            wait_wb(b, j - _NBUF)
            transpose(b)
            start_wb(b, j)
            start_gather(b, j + _NBUF)
        return carry

    lax.fori_loop(1, _NROUND - 1, round_body, 0)

    gl = _NROUND - 1
    for b in range(_NBUF):
        j = gl * _NBUF + b
        wait_gather(b, j)
        wait_wb(b, j - _NBUF)
        transpose(b)
        start_wb(b, j)
    for b in range(_NBUF):
        wait_wb(b, gl * _NBUF + b)


def kernel(input_variable, embedding_weight):
    idx = input_variable
    if idx.dtype != jnp.int32:
        idx = idx.astype(jnp.int32)
    table128 = _transpose_pad(embedding_weight.T)
    out3d = _embed_sc(idx.T, table128)
    return jnp.transpose(out3d, (2, 0, 1))


# speedup vs baseline: 1.4490x; 1.3401x over previous
import functools

import jax
import jax.numpy as jnp
from jax import lax
from jax.experimental import pallas as pl
from jax.experimental.pallas import tpu as pltpu
from jax.experimental.pallas import tpu_sc as plsc

_VOCAB = 1000000
_EMSIZE = 64
_PADE = 128
_B = 4096
_L = 50

_NC = 2
_NS = 16
_NW = _NC * _NS
_RPW = _B // _NW
_NBUF = 4
_NT = 2

_mesh = plsc.VectorSubcoreMesh(core_axis_name="c", subcore_axis_name="s")

_TBS = 4096
_TGRID = -(-_VOCAB // _TBS)


def _tpose_body(t_ref, o_ref):
    eye = jnp.eye(_EMSIZE, dtype=jnp.float32)
    t = jax.lax.dot_general(
        t_ref[...], eye, (((0,), (0,)), ((), ())),
        preferred_element_type=jnp.float32)
    o_ref[...] = jnp.concatenate(
        [t, jnp.zeros((_TBS, _PADE - _EMSIZE), jnp.float32)], axis=1)


_transpose_pad = pl.pallas_call(
    _tpose_body,
    grid=(_TGRID,),
    in_specs=[pl.BlockSpec((_EMSIZE, _TBS), lambda k: (0, k))],
    out_specs=pl.BlockSpec((_TBS, _PADE), lambda k: (k, 0)),
    out_shape=jax.ShapeDtypeStruct((_VOCAB, _PADE), jnp.float32),
)


@functools.partial(
    pl.kernel,
    mesh=_mesh,
    compiler_params=pltpu.CompilerParams(
        use_tc_tiling_on_sc=False, needs_layout_passes=False),
    out_type=jax.ShapeDtypeStruct((_L, _EMSIZE, _B), jnp.float32),
    scratch_types=(
        [pltpu.VMEM((_L, _RPW), jnp.int32)]
        + [pltpu.VMEM((_RPW, _PADE), jnp.float32)] * _NBUF
        + [pltpu.VMEM((_EMSIZE, _RPW + 1), jnp.float32)] * _NT
        + [pltpu.SemaphoreType.DMA] * (_NBUF + _NT)
    ),
)
def _embed_sc(idx_hbm, table_hbm, out_hbm, idx_v, *rest):
    bufs = rest[:_NBUF]
    bufTs = rest[_NBUF:_NBUF + _NT]
    gs = rest[_NBUF + _NT:2 * _NBUF + _NT]
    ws = rest[2 * _NBUF + _NT:]
    wid = lax.axis_index("s") * _NC + lax.axis_index("c")
    base = wid * _RPW
    pltpu.sync_copy(idx_hbm.at[:, pl.ds(base, _RPW)], idx_v)

    def start_gather(b, j):
        pltpu.async_copy(table_hbm.at[idx_v.at[j]], bufs[b], gs[b])

    def wait_gather(b, j):
        pltpu.make_async_copy(
            table_hbm.at[idx_v.at[j]], bufs[b], gs[b]).wait()

    def start_wb(t, j):
        pltpu.async_copy(
            bufTs[t].at[:, pl.ds(0, _RPW)],
            out_hbm.at[j, :, pl.ds(base, _RPW)], ws[t])

    def wait_wb(t, j):
        pltpu.make_async_copy(
            bufTs[t].at[:, pl.ds(0, _RPW)],
            out_hbm.at[j, :, pl.ds(base, _RPW)], ws[t]).wait()

    riotas = [lax.iota(jnp.int32, 16) + 16 * q for q in range(_EMSIZE // 16)]

    def transpose(b, t):
        def rbody(ri, carry):
            for rr in range(4):
                r = ri * 4 + rr
                col = jnp.full((16,), r, jnp.int32)
                vs = [bufs[b][r, pl.ds(16 * q, 16)]
                      for q in range(_EMSIZE // 16)]
                for q in range(_EMSIZE // 16):
                    plsc.store_scatter(bufTs[t], [riotas[q], col], vs[q])
            return carry
        lax.fori_loop(0, _RPW // 4, rbody, 0)

    def do_chunk(b, j, wait_prev):
        t = b % _NT
        wait_gather(b, j)
        if wait_prev:
            wait_wb(t, j - _NT)
        transpose(b, t)
        start_wb(t, j)

    for b in range(_NBUF):
        start_gather(b, b)
    for b in range(_NBUF):
        do_chunk(b, b, wait_prev=(b >= _NT))
        start_gather(b, b + _NBUF)

    def round_body(g, carry):
        for b in range(_NBUF):
            j = g * _NBUF + b
            do_chunk(b, j, True)
            start_gather(b, j + _NBUF)
        return carry

    lax.fori_loop(1, 11, round_body, 0)

    for b in range(_NBUF):
        do_chunk(b, 44 + b, True)
        if b < _L - 48:
            start_gather(b, 48 + b)
    for b in range(_L - 48):
        do_chunk(b, 48 + b, True)
    for b in range(_L - 48):
        wait_wb(b % _NT, 48 + b)


def kernel(input_variable, embedding_weight):
    idx = input_variable
    if idx.dtype != jnp.int32:
        idx = idx.astype(jnp.int32)
    table128 = _transpose_pad(embedding_weight.T)
    out3d = _embed_sc(idx.T, table128)
    return jnp.transpose(out3d, (2, 0, 1))
